# baseline (speedup 1.0000x reference)
import functools

import jax
import jax.numpy as jnp
from jax import lax
from jax.experimental import pallas as pl
from jax.experimental.pallas import tpu as pltpu
from jax.experimental.pallas import tpu_sc as plsc

D_MODEL = 1024
BATCH = 4
SEQ_LEN = 4096
B_TOTAL = BATCH * SEQ_LEN

_INFO = plsc.get_sparse_core_info()
NC = _INFO.num_cores
NS = _INFO.num_subcores
NW = NC * NS
B_PER_W = B_TOTAL // NW
W_PER_ROW = SEQ_LEN // B_PER_W
CHUNK = 16
N_CHUNKS = B_PER_W // CHUNK
DEPTH = 4

_MESH = plsc.VectorSubcoreMesh(core_axis_name="c", subcore_axis_name="s")


@functools.partial(
    pl.kernel,
    mesh=_MESH,
    out_type=jax.ShapeDtypeStruct((BATCH, SEQ_LEN, D_MODEL), jnp.float32),
    scratch_types=(
        [pltpu.VMEM((B_PER_W,), jnp.int32)]
        + [pltpu.VMEM((CHUNK, D_MODEL), jnp.float32)] * DEPTH
        + [pltpu.SemaphoreType.DMA] * (2 * DEPTH)
    ),
)
def _sc_gather(idx_hbm, table_hbm, out_hbm, idx_v, *rest):
    bufs = rest[:DEPTH]
    gsems = rest[DEPTH:2 * DEPTH]
    ssems = rest[2 * DEPTH:]
    wid = lax.axis_index("s") * NC + lax.axis_index("c")
    row = wid // W_PER_ROW
    col = (wid % W_PER_ROW) * B_PER_W
    pltpu.sync_copy(idx_hbm.at[row, pl.ds(col, B_PER_W)], idx_v)
    gath = [None] * DEPTH
    scat = [None] * DEPTH

    def gather(j):
        b = j % DEPTH
        if scat[b] is not None:
            scat[b].wait()
        gath[b] = pltpu.async_copy(
            table_hbm.at[idx_v.at[pl.ds(j * CHUNK, CHUNK)]],
            bufs[b], gsems[b])

    def scatter(j):
        b = j % DEPTH
        gath[b].wait()
        scat[b] = pltpu.async_copy(
            bufs[b], out_hbm.at[row, pl.ds(col + j * CHUNK, CHUNK)], ssems[b])

    for j in range(N_CHUNKS):
        gather(j)
        if j >= DEPTH - 1:
            scatter(j - (DEPTH - 1))
    for j in range(N_CHUNKS - DEPTH + 1, N_CHUNKS):
        scatter(j)
    for b in range(DEPTH):
        scat[b].wait()


def kernel(x, table):
    return _sc_gather(x.astype(jnp.int32), table)


# speedup vs baseline: 1.0280x; 1.0280x over previous
import functools

import jax
import jax.numpy as jnp
from jax import lax
from jax.experimental import pallas as pl
from jax.experimental.pallas import tpu as pltpu
from jax.experimental.pallas import tpu_sc as plsc

D_MODEL = 1024
BATCH = 4
SEQ_LEN = 4096
B_TOTAL = BATCH * SEQ_LEN

_INFO = plsc.get_sparse_core_info()
NC = _INFO.num_cores
NS = _INFO.num_subcores
NW = NC * NS
B_PER_W = B_TOTAL // NW
W_PER_ROW = SEQ_LEN // B_PER_W
CHUNK = 16
N_CHUNKS = B_PER_W // CHUNK
DEPTH = 4

_MESH = plsc.VectorSubcoreMesh(core_axis_name="c", subcore_axis_name="s")


@functools.partial(
    pl.kernel,
    mesh=_MESH,
    out_type=jax.ShapeDtypeStruct((BATCH, SEQ_LEN, D_MODEL), jnp.float32),
    scratch_types=(
        [pltpu.VMEM((B_PER_W,), jnp.int32)]
        + [pltpu.VMEM((CHUNK, D_MODEL), jnp.float32)] * DEPTH
        + [pltpu.SemaphoreType.DMA] * (2 * DEPTH)
    ),
)
def _sc_gather(idx_hbm, table_hbm, out_hbm, idx_v, *rest):
    bufs = rest[:DEPTH]
    gsems = rest[DEPTH:2 * DEPTH]
    ssems = rest[2 * DEPTH:]
    wid = lax.axis_index("s") * NC + lax.axis_index("c")
    row = wid // W_PER_ROW
    col = (wid % W_PER_ROW) * B_PER_W
    pltpu.sync_copy(idx_hbm.at[row, pl.ds(col, B_PER_W)], idx_v)

    def start_gather(j, b):
        pltpu.async_copy(
            table_hbm.at[idx_v.at[pl.ds(j * CHUNK, CHUNK)]],
            bufs[b], gsems[b])

    def wait_gather(b):
        pltpu.make_async_copy(
            table_hbm.at[pl.ds(0, CHUNK)], bufs[b], gsems[b]).wait()

    def start_scatter(j, b):
        pltpu.async_copy(
            bufs[b], out_hbm.at[row, pl.ds(col + j * CHUNK, CHUNK)], ssems[b])

    def wait_scatter(j, b):
        pltpu.make_async_copy(
            bufs[b], out_hbm.at[row, pl.ds(col + j * CHUNK, CHUNK)],
            ssems[b]).wait()

    for j in range(DEPTH):
        start_gather(j, j)
    wait_gather(0)
    start_scatter(0, 0)

    @pl.loop(1, N_CHUNKS // DEPTH)
    def _steady(g):
        for b in range(DEPTH):
            j = g * DEPTH + b
            wait_scatter(j - DEPTH, b)
            start_gather(j, b)
            bp = (b + 1) % DEPTH
            wait_gather(bp)
            start_scatter(j - (DEPTH - 1), bp)

    for j in range(N_CHUNKS - DEPTH + 1, N_CHUNKS):
        b = j % DEPTH
        wait_gather(b)
        start_scatter(j, b)
    for j in range(N_CHUNKS - DEPTH, N_CHUNKS):
        wait_scatter(j, j % DEPTH)


def kernel(x, table):
    return _sc_gather(x.astype(jnp.int32), table)
